# baseline (speedup 1.0000x reference)
import functools

import jax
import jax.numpy as jnp
from jax import lax
from jax.experimental import pallas as pl
from jax.experimental.pallas import tpu as pltpu
from jax.experimental.pallas import tpu_sc as plsc

SEQ = 4096
BATCH = 4
D_MODEL = 1024
NC = 2
NS = 16
NW = NC * NS

B = SEQ * BATCH
B_PER_W = B // NW
CHUNK = 32
SEQ_PER_CHUNK = CHUNK // BATCH
N_CHUNKS = B_PER_W // CHUNK
LANES = 16
VECS = D_MODEL // LANES


def _sc_body(x_hbm, emb_hbm, pe_hbm, out_hbm, idx_v, rows_v, pe_v, gsem, psem, osem):
    wid = lax.axis_index("s") * NC + lax.axis_index("c")
    base = wid * B_PER_W
    seq_base = wid * (B_PER_W // BATCH)

    pltpu.sync_copy(x_hbm.at[wid], idx_v)

    def chunk_body(g, carry):
        gcopy = pltpu.async_copy(emb_hbm.at[idx_v.at[g]], rows_v, gsem)
        pcopy = pltpu.async_copy(
            pe_hbm.at[pl.ds(seq_base + g * SEQ_PER_CHUNK, SEQ_PER_CHUNK)], pe_v, psem
        )
        gcopy.wait()
        pcopy.wait()

        def seq_body(s, c2):
            def vec_body(j, c3):
                off = j * LANES
                pv = pe_v[s, pl.ds(off, LANES)]
                row0 = s * BATCH
                for b in range(BATCH):
                    rows_v[row0 + b, pl.ds(off, LANES)] += pv
                return c3

            return lax.fori_loop(0, VECS, vec_body, c2)

        lax.fori_loop(0, SEQ_PER_CHUNK, seq_body, 0)

        pltpu.async_copy(rows_v, out_hbm.at[pl.ds(base + g * CHUNK, CHUNK)], osem).wait()
        return carry

    lax.fori_loop(0, N_CHUNKS, chunk_body, 0)


def kernel(x, emb, pe):
    seq, batch = x.shape
    x_grp = x.reshape(NW, N_CHUNKS, CHUNK)
    pe2d = pe[:seq, 0, :]

    mesh = plsc.VectorSubcoreMesh(core_axis_name="c", subcore_axis_name="s")
    run = functools.partial(
        pl.kernel,
        mesh=mesh,
        out_type=jax.ShapeDtypeStruct((B, D_MODEL), jnp.float32),
        scratch_types=[
            pltpu.VMEM((N_CHUNKS, CHUNK), jnp.int32),
            pltpu.VMEM((CHUNK, D_MODEL), jnp.float32),
            pltpu.VMEM((SEQ_PER_CHUNK, D_MODEL), jnp.float32),
            pltpu.SemaphoreType.DMA,
            pltpu.SemaphoreType.DMA,
            pltpu.SemaphoreType.DMA,
        ],
    )(_sc_body)
    out = run(x_grp, emb, pe2d)
    return out.reshape(seq, batch, D_MODEL)


# speedup vs baseline: 1.2693x; 1.2693x over previous
import functools

import jax
import jax.numpy as jnp
from jax import lax
from jax.experimental import pallas as pl
from jax.experimental.pallas import tpu as pltpu
from jax.experimental.pallas import tpu_sc as plsc

SEQ = 4096
BATCH = 4
D_MODEL = 1024
NC = 2
NS = 16
NW = NC * NS

B = SEQ * BATCH
B_PER_W = B // NW
CHUNK = 32
SEQ_PER_CHUNK = CHUNK // BATCH
N_CHUNKS = B_PER_W // CHUNK
SEQ_PER_W = B_PER_W // BATCH
LANES = 16
VECS = D_MODEL // LANES


def _sc_body(x_hbm, emb_hbm, pe_hbm, out_hbm, idx_v, rows_v, pe_v, gsem, psem, osem):
    wid = lax.axis_index("s") * NC + lax.axis_index("c")
    seq_base = wid * SEQ_PER_W

    pltpu.sync_copy(x_hbm.at[wid], idx_v)

    def chunk_body(g, carry):
        sq = seq_base + g * SEQ_PER_CHUNK
        gcopy = pltpu.async_copy(emb_hbm.at[idx_v.at[g]], rows_v, gsem)
        pcopy = pltpu.async_copy(pe_hbm.at[pl.ds(sq, SEQ_PER_CHUNK)], pe_v, psem)
        gcopy.wait()
        pcopy.wait()

        def seq_body(s, c2):
            row0 = s * BATCH
            for j in range(VECS):
                off = j * LANES
                pv = pe_v[s, 0, pl.ds(off, LANES)]
                for b in range(BATCH):
                    rows_v[row0 + b, pl.ds(off, LANES)] += pv
            return c2

        lax.fori_loop(0, SEQ_PER_CHUNK, seq_body, 0)

        copies = [
            pltpu.async_copy(
                rows_v.at[pl.ds(s * BATCH, BATCH)], out_hbm.at[sq + s], osem
            )
            for s in range(SEQ_PER_CHUNK)
        ]
        for c in copies:
            c.wait()
        return carry

    lax.fori_loop(0, N_CHUNKS, chunk_body, 0)


def kernel(x, emb, pe):
    seq, batch = x.shape
    x_grp = x.reshape(NW, N_CHUNKS, CHUNK)

    mesh = plsc.VectorSubcoreMesh(core_axis_name="c", subcore_axis_name="s")
    run = functools.partial(
        pl.kernel,
        mesh=mesh,
        out_type=jax.ShapeDtypeStruct((SEQ, BATCH, D_MODEL), jnp.float32),
        scratch_types=[
            pltpu.VMEM((N_CHUNKS, CHUNK), jnp.int32),
            pltpu.VMEM((CHUNK, D_MODEL), jnp.float32),
            pltpu.VMEM((SEQ_PER_CHUNK, 1, D_MODEL), jnp.float32),
            pltpu.SemaphoreType.DMA,
            pltpu.SemaphoreType.DMA,
            pltpu.SemaphoreType.DMA,
        ],
    )(_sc_body)
    return run(x_grp, emb, pe)
